# baseline (speedup 1.0000x reference)
import functools

import jax
import jax.numpy as jnp
from jax import lax
from jax.experimental import pallas as pl
from jax.experimental.pallas import tpu as pltpu
from jax.experimental.pallas import tpu_sc as plsc

_L = 16
_NC = 2
_NS = 16
_NW = _NC * _NS

_ITERS = 20
_NEG = -3e38


@functools.lru_cache(maxsize=None)
def _build(n_rows, n_cols):
  assert n_rows % _NW == 0 and n_cols % _L == 0
  rows_per_w = n_rows // _NW
  ntiles = n_cols // _L
  mesh = plsc.VectorSubcoreMesh(core_axis_name="c", subcore_axis_name="s")

  @functools.partial(
      pl.kernel,
      out_type=jax.ShapeDtypeStruct((n_rows, n_cols), jnp.float32),
      mesh=mesh,
      compiler_params=pltpu.CompilerParams(needs_layout_passes=False),
      scratch_types=[
          pltpu.VMEM((n_cols,), jnp.float32),
          pltpu.VMEM((n_cols,), jnp.float32),
          pltpu.VMEM((n_cols,), jnp.float32),
          pltpu.SemaphoreType.DMA,
          pltpu.SemaphoreType.DMA,
          pltpu.SemaphoreType.DMA,
          pltpu.SemaphoreType.DMA,
      ],
  )
  def sparsemax_kernel(x_hbm, out_hbm, row_a, row_b, cand_v,
                       isem_a, isem_b, osem_a, osem_b):
    wid = lax.axis_index("s") * _NC + lax.axis_index("c")
    base = wid * rows_per_w
    bufs = (row_a, row_b)
    isems = (isem_a, isem_b)
    osems = (osem_a, osem_b)

    def in_cp(r, b):
      return pltpu.make_async_copy(x_hbm.at[base + r], bufs[b], isems[b])

    def out_cp(r, b):
      return pltpu.make_async_copy(bufs[b], out_hbm.at[base + r], osems[b])

    def do_row(row_v):
      @plsc.parallel_loop(0, n_cols, step=_L, unroll=8,
                          carry=jnp.full((_L,), _NEG, jnp.float32))
      def acc(i, a):
        return jnp.maximum(a, row_v[pl.ds(i, _L)])
      m = jnp.max(acc)
      t0 = m - 1.0

      col0 = lax.iota(jnp.int32, _L) * ntiles

      @plsc.parallel_loop(0, n_cols, step=_L, unroll=8,
                          carry=jnp.zeros((_L,), jnp.int32))
      def cnt(i, c):
        v = row_v[pl.ds(i, _L)]
        msk = v > t0
        plsc.store_scatter(cand_v, [col0 + c], v, mask=msk)
        return c + jnp.where(msk, 1, 0)
      ctiles = jnp.max(cnt)

      def pad_body(s, carry2):
        plsc.store_scatter(cand_v, [col0 + s],
                           jnp.full((_L,), _NEG, jnp.float32),
                           mask=cnt <= s)
        return carry2
      lax.fori_loop(0, ctiles, pad_body, 0)

      def newton_cond(st):
        it, tau, prev = st[0], st[1], st[2]
        return jnp.logical_and(it < _ITERS, jnp.any(tau != prev))

      def newton(st):
        it, tau, _, lo, hi, best = st
        def acc_body(i, sc):
          s16, c16 = sc
          d = plsc.load_gather(cand_v, [col0 + i]) - tau
          msk = d > 0.0
          return (s16 + jnp.maximum(d, 0.0), c16 + jnp.where(msk, 1.0, 0.0))
        s16, c16 = lax.fori_loop(
            0, ctiles, acc_body,
            (jnp.zeros((_L,), jnp.float32), jnp.zeros((_L,), jnp.float32)))
        f = jnp.full((_L,), jnp.sum(s16) - 1.0, jnp.float32)
        c = jnp.full((_L,), jnp.maximum(jnp.sum(c16), 1.0), jnp.float32)
        nt = tau + f / c
        left = f >= 0.0
        lo = jnp.where(left, tau, lo)
        hi = jnp.where(left, hi, tau)
        best = jnp.where(left, nt, best)
        mid = 0.5 * (lo + hi)
        good = (nt > lo) & (nt < hi)
        return (it + 1, jnp.where(good, nt, mid), tau, lo, hi, best)

      t0v = jnp.full((_L,), t0, jnp.float32)
      mv = jnp.full((_L,), m, jnp.float32)
      tau = lax.while_loop(
          newton_cond, newton,
          (jnp.int32(0), t0v, t0v - 1.0, t0v, mv, t0v))[5]

      @plsc.parallel_loop(0, n_cols, step=_L, unroll=8)
      def _(i):
        sl = pl.ds(i, _L)
        row_v[sl] = jnp.maximum(row_v[sl] - tau, 0.0)

    in_cp(0, 0).start()
    for r in range(rows_per_w):
      b = r % 2
      if r + 1 < rows_per_w:
        if r >= 1:
          out_cp(r - 1, 1 - b).wait()
        in_cp(r + 1, 1 - b).start()
      in_cp(r, b).wait()
      do_row(bufs[b])
      out_cp(r, b).start()
    if rows_per_w >= 2:
      out_cp(rows_per_w - 2, rows_per_w % 2).wait()
    out_cp(rows_per_w - 1, (rows_per_w - 1) % 2).wait()

  return sparsemax_kernel


def kernel(logits):
  n_rows, n_cols = logits.shape
  return _build(n_rows, n_cols)(logits)


# speedup vs baseline: 1.0659x; 1.0659x over previous
import functools

import jax
import jax.numpy as jnp
from jax import lax
from jax.experimental import pallas as pl
from jax.experimental.pallas import tpu as pltpu
from jax.experimental.pallas import tpu_sc as plsc

_L = 16
_NC = 2
_NS = 16
_NW = _NC * _NS

_ITERS = 20
_NEG = -3e38


@functools.lru_cache(maxsize=None)
def _build(n_rows, n_cols):
  assert n_rows % _NW == 0 and n_cols % _L == 0
  rows_per_w = n_rows // _NW
  ntiles = n_cols // _L
  mesh = plsc.VectorSubcoreMesh(core_axis_name="c", subcore_axis_name="s")

  @functools.partial(
      pl.kernel,
      out_type=jax.ShapeDtypeStruct((n_rows, n_cols), jnp.float32),
      mesh=mesh,
      compiler_params=pltpu.CompilerParams(needs_layout_passes=False),
      scratch_types=[
          pltpu.VMEM((n_cols,), jnp.float32),
          pltpu.VMEM((n_cols,), jnp.float32),
          pltpu.VMEM((n_cols,), jnp.float32),
          pltpu.SemaphoreType.DMA,
          pltpu.SemaphoreType.DMA,
          pltpu.SemaphoreType.DMA,
          pltpu.SemaphoreType.DMA,
      ],
  )
  def sparsemax_kernel(x_hbm, out_hbm, row_a, row_b, cand_v,
                       isem_a, isem_b, osem_a, osem_b):
    wid = lax.axis_index("s") * _NC + lax.axis_index("c")
    base = wid * rows_per_w
    bufs = (row_a, row_b)
    isems = (isem_a, isem_b)
    osems = (osem_a, osem_b)

    def in_cp(r, b):
      return pltpu.make_async_copy(x_hbm.at[base + r], bufs[b], isems[b])

    def out_cp(r, b):
      return pltpu.make_async_copy(bufs[b], out_hbm.at[base + r], osems[b])

    def do_row(row_v):
      @plsc.parallel_loop(0, n_cols, step=_L, unroll=8,
                          carry=jnp.full((_L,), _NEG, jnp.float32))
      def acc(i, a):
        return jnp.maximum(a, row_v[pl.ds(i, _L)])
      m = jnp.max(acc)
      t0 = m - 1.0

      iota = lax.iota(jnp.int32, _L)

      @plsc.parallel_loop(0, n_cols, step=_L, unroll=8,
                          carry=jnp.zeros((_L,), jnp.int32))
      def cnt(i, c):
        v = row_v[pl.ds(i, _L)]
        msk = v > t0
        plsc.store_scatter(cand_v, [c * _L + iota], v, mask=msk)
        return c + jnp.where(msk, 1, 0)
      ctiles = jnp.max(cnt)

      def pad_body(s, carry2):
        plsc.store_scatter(cand_v, [s * _L + iota],
                           jnp.full((_L,), _NEG, jnp.float32),
                           mask=cnt <= s)
        return carry2
      lax.fori_loop(0, ctiles, pad_body, 0)

      def newton_cond(st):
        it, tau, prev = st[0], st[1], st[2]
        return jnp.logical_and(it < _ITERS, jnp.any(tau != prev))

      def newton(st):
        it, tau, _, lo, hi, best = st
        def acc_body(i, sc):
          s16, c16 = sc
          d = cand_v[pl.ds(i * _L, _L)] - tau
          msk = d > 0.0
          return (s16 + jnp.maximum(d, 0.0), c16 + jnp.where(msk, 1.0, 0.0))
        s16, c16 = lax.fori_loop(
            0, ctiles, acc_body,
            (jnp.zeros((_L,), jnp.float32), jnp.zeros((_L,), jnp.float32)))
        f = jnp.full((_L,), jnp.sum(s16) - 1.0, jnp.float32)
        c = jnp.full((_L,), jnp.maximum(jnp.sum(c16), 1.0), jnp.float32)
        nt = tau + f / c
        left = f >= 0.0
        lo = jnp.where(left, tau, lo)
        hi = jnp.where(left, hi, tau)
        best = jnp.where(left, nt, best)
        mid = 0.5 * (lo + hi)
        good = (nt > lo) & (nt < hi)
        return (it + 1, jnp.where(good, nt, mid), tau, lo, hi, best)

      t0v = jnp.full((_L,), t0, jnp.float32)
      mv = jnp.full((_L,), m, jnp.float32)
      tau = lax.while_loop(
          newton_cond, newton,
          (jnp.int32(0), t0v, t0v - 1.0, t0v, mv, t0v))[5]

      @plsc.parallel_loop(0, n_cols, step=_L, unroll=8)
      def _(i):
        sl = pl.ds(i, _L)
        row_v[sl] = jnp.maximum(row_v[sl] - tau, 0.0)

    in_cp(0, 0).start()
    for r in range(rows_per_w):
      b = r % 2
      if r + 1 < rows_per_w:
        if r >= 1:
          out_cp(r - 1, 1 - b).wait()
        in_cp(r + 1, 1 - b).start()
      in_cp(r, b).wait()
      do_row(bufs[b])
      out_cp(r, b).start()
    if rows_per_w >= 2:
      out_cp(rows_per_w - 2, rows_per_w % 2).wait()
    out_cp(rows_per_w - 1, (rows_per_w - 1) % 2).wait()

  return sparsemax_kernel


def kernel(logits):
  n_rows, n_cols = logits.shape
  return _build(n_rows, n_cols)(logits)


# speedup vs baseline: 1.1404x; 1.0699x over previous
import functools

import jax
import jax.numpy as jnp
from jax import lax
from jax.experimental import pallas as pl
from jax.experimental.pallas import tpu as pltpu
from jax.experimental.pallas import tpu_sc as plsc

_L = 16
_NC = 2
_NS = 16
_NW = _NC * _NS

_ITERS = 12
_NEG = -3e38


@functools.lru_cache(maxsize=None)
def _build(n_rows, n_cols):
  assert n_rows % _NW == 0 and n_cols % _L == 0
  rows_per_w = n_rows // _NW
  ntiles = n_cols // _L
  mesh = plsc.VectorSubcoreMesh(core_axis_name="c", subcore_axis_name="s")

  @functools.partial(
      pl.kernel,
      out_type=jax.ShapeDtypeStruct((n_rows, n_cols), jnp.float32),
      mesh=mesh,
      compiler_params=pltpu.CompilerParams(needs_layout_passes=False),
      scratch_types=[
          pltpu.VMEM((n_cols,), jnp.float32),
          pltpu.VMEM((n_cols,), jnp.float32),
          pltpu.VMEM((n_cols,), jnp.float32),
          pltpu.SemaphoreType.DMA,
          pltpu.SemaphoreType.DMA,
          pltpu.SemaphoreType.DMA,
          pltpu.SemaphoreType.DMA,
      ],
  )
  def sparsemax_kernel(x_hbm, out_hbm, row_a, row_b, cand_v,
                       isem_a, isem_b, osem_a, osem_b):
    wid = lax.axis_index("s") * _NC + lax.axis_index("c")
    base = wid * rows_per_w
    bufs = (row_a, row_b)
    isems = (isem_a, isem_b)
    osems = (osem_a, osem_b)

    def in_cp(r, b):
      return pltpu.make_async_copy(x_hbm.at[base + r], bufs[b], isems[b])

    def out_cp(r, b):
      return pltpu.make_async_copy(bufs[b], out_hbm.at[base + r], osems[b])

    def do_row(row_v):
      @plsc.parallel_loop(0, n_cols, step=_L, unroll=8,
                          carry=jnp.full((_L,), _NEG, jnp.float32))
      def acc(i, a):
        return jnp.maximum(a, row_v[pl.ds(i, _L)])
      m = jnp.max(acc)
      t0 = m - 1.0

      iota = lax.iota(jnp.int32, _L)

      @plsc.parallel_loop(0, n_cols, step=_L, unroll=8,
                          carry=jnp.zeros((_L,), jnp.int32))
      def cnt(i, c):
        v = row_v[pl.ds(i, _L)]
        msk = v > t0
        plsc.store_scatter(cand_v, [c * _L + iota], v, mask=msk)
        return c + jnp.where(msk, 1, 0)
      ctiles = jnp.max(cnt)

      def pad_body(s, carry2):
        plsc.store_scatter(cand_v, [s * _L + iota],
                           jnp.full((_L,), _NEG, jnp.float32),
                           mask=cnt <= s)
        return carry2
      lax.fori_loop(0, ctiles, pad_body, 0)

      def newton(it, st):
        tau, lo, hi, best = st
        def acc_body(i, sc):
          s16, c16 = sc
          d = cand_v[pl.ds(i * _L, _L)] - tau
          msk = d > 0.0
          return (s16 + jnp.maximum(d, 0.0), c16 + jnp.where(msk, 1.0, 0.0))
        s16, c16 = lax.fori_loop(
            0, ctiles, acc_body,
            (jnp.zeros((_L,), jnp.float32), jnp.zeros((_L,), jnp.float32)))
        f = jnp.full((_L,), jnp.sum(s16) - 1.0, jnp.float32)
        c = jnp.full((_L,), jnp.maximum(jnp.sum(c16), 1.0), jnp.float32)
        nt = tau + f / c
        left = f >= 0.0
        lo = jnp.where(left, tau, lo)
        hi = jnp.where(left, hi, tau)
        best = jnp.where(left, nt, best)
        mid = 0.5 * (lo + hi)
        good = (nt > lo) & (nt < hi)
        return (jnp.where(good, nt, mid), lo, hi, best)

      t0v = jnp.full((_L,), t0, jnp.float32)
      mv = jnp.full((_L,), m, jnp.float32)
      tau = lax.fori_loop(0, _ITERS, newton, (t0v, t0v, mv, t0v))[3]

      @plsc.parallel_loop(0, n_cols, step=_L, unroll=8)
      def _(i):
        sl = pl.ds(i, _L)
        row_v[sl] = jnp.maximum(row_v[sl] - tau, 0.0)

    in_cp(0, 0).start()
    for r in range(rows_per_w):
      b = r % 2
      if r + 1 < rows_per_w:
        if r >= 1:
          out_cp(r - 1, 1 - b).wait()
        in_cp(r + 1, 1 - b).start()
      in_cp(r, b).wait()
      do_row(bufs[b])
      out_cp(r, b).start()
    if rows_per_w >= 2:
      out_cp(rows_per_w - 2, rows_per_w % 2).wait()
    out_cp(rows_per_w - 1, (rows_per_w - 1) % 2).wait()

  return sparsemax_kernel


def kernel(logits):
  n_rows, n_cols = logits.shape
  return _build(n_rows, n_cols)(logits)


# speedup vs baseline: 1.1881x; 1.0419x over previous
import functools

import jax
import jax.numpy as jnp
from jax import lax
from jax.experimental import pallas as pl
from jax.experimental.pallas import tpu as pltpu
from jax.experimental.pallas import tpu_sc as plsc

_L = 16
_NC = 2
_NS = 16
_NW = _NC * _NS

_ITERS = 12
_NEG = -3e38


@functools.lru_cache(maxsize=None)
def _build(n_rows, n_cols):
  assert n_rows % _NW == 0 and n_cols % _L == 0
  rows_per_w = n_rows // _NW
  ntiles = n_cols // _L
  mesh = plsc.VectorSubcoreMesh(core_axis_name="c", subcore_axis_name="s")

  @functools.partial(
      pl.kernel,
      out_type=jax.ShapeDtypeStruct((n_rows, n_cols), jnp.float32),
      mesh=mesh,
      compiler_params=pltpu.CompilerParams(needs_layout_passes=False),
      scratch_types=[
          pltpu.VMEM((n_cols,), jnp.float32),
          pltpu.VMEM((n_cols,), jnp.float32),
          pltpu.VMEM((n_cols,), jnp.float32),
          pltpu.SemaphoreType.DMA,
          pltpu.SemaphoreType.DMA,
          pltpu.SemaphoreType.DMA,
          pltpu.SemaphoreType.DMA,
      ],
  )
  def sparsemax_kernel(x_hbm, out_hbm, row_a, row_b, cand_v,
                       isem_a, isem_b, osem_a, osem_b):
    wid = lax.axis_index("s") * _NC + lax.axis_index("c")
    base = wid * rows_per_w
    bufs = (row_a, row_b)
    isems = (isem_a, isem_b)
    osems = (osem_a, osem_b)

    def in_cp(r, b):
      return pltpu.make_async_copy(x_hbm.at[base + r], bufs[b], isems[b])

    def out_cp(r, b):
      return pltpu.make_async_copy(bufs[b], out_hbm.at[base + r], osems[b])

    def do_row(row_v):
      @plsc.parallel_loop(0, n_cols, step=_L, unroll=8,
                          carry=jnp.full((_L,), _NEG, jnp.float32))
      def acc(i, a):
        return jnp.maximum(a, row_v[pl.ds(i, _L)])
      m = jnp.max(acc)
      t0 = m - 1.0

      iota = lax.iota(jnp.int32, _L)

      @plsc.parallel_loop(0, n_cols, step=_L, unroll=8,
                          carry=jnp.zeros((_L,), jnp.int32))
      def cnt16(i, c):
        v = row_v[pl.ds(i, _L)]
        msk = v > t0
        plsc.store_scatter(cand_v, [c | iota], v, mask=msk)
        return c + jnp.where(msk, _L, 0)
      ctiles = jnp.max(cnt16) // _L

      def pad_body(s, carry2):
        plsc.store_scatter(cand_v, [s * _L + iota],
                           jnp.full((_L,), _NEG, jnp.float32),
                           mask=cnt16 <= s * _L)
        return carry2
      lax.fori_loop(0, ctiles, pad_body, 0)

      def newton(it, st):
        tau, lo, hi, best = st
        def acc_body(i, sc):
          s16, c16 = sc
          d = cand_v[pl.ds(i * _L, _L)] - tau
          msk = d > 0.0
          return (s16 + jnp.maximum(d, 0.0), c16 + jnp.where(msk, 1.0, 0.0))
        s16, c16 = lax.fori_loop(
            0, ctiles, acc_body,
            (jnp.zeros((_L,), jnp.float32), jnp.zeros((_L,), jnp.float32)))
        f = jnp.full((_L,), jnp.sum(s16) - 1.0, jnp.float32)
        c = jnp.full((_L,), jnp.maximum(jnp.sum(c16), 1.0), jnp.float32)
        nt = tau + f / c
        left = f >= 0.0
        lo = jnp.where(left, tau, lo)
        hi = jnp.where(left, hi, tau)
        best = jnp.where(left, nt, best)
        mid = 0.5 * (lo + hi)
        good = (nt > lo) & (nt < hi)
        return (jnp.where(good, nt, mid), lo, hi, best)

      t0v = jnp.full((_L,), t0, jnp.float32)
      mv = jnp.full((_L,), m, jnp.float32)
      tau = lax.fori_loop(0, _ITERS, newton, (t0v, t0v, mv, t0v))[3]

      @plsc.parallel_loop(0, n_cols, step=_L, unroll=8)
      def _(i):
        sl = pl.ds(i, _L)
        row_v[sl] = jnp.maximum(row_v[sl] - tau, 0.0)

    in_cp(0, 0).start()
    for r in range(rows_per_w):
      b = r % 2
      if r + 1 < rows_per_w:
        if r >= 1:
          out_cp(r - 1, 1 - b).wait()
        in_cp(r + 1, 1 - b).start()
      in_cp(r, b).wait()
      do_row(bufs[b])
      out_cp(r, b).start()
    if rows_per_w >= 2:
      out_cp(rows_per_w - 2, rows_per_w % 2).wait()
    out_cp(rows_per_w - 1, (rows_per_w - 1) % 2).wait()

  return sparsemax_kernel


def kernel(logits):
  n_rows, n_cols = logits.shape
  return _build(n_rows, n_cols)(logits)
